# baseline (device time: 12557 ns/iter reference)
import jax
import jax.numpy as jnp
from jax import lax
from jax.experimental import pallas as pl
from jax.experimental.pallas import tpu as pltpu

N_CHUNKS = 4


def kernel(x, pi):
    _, m, n = x.shape
    rows = m // N_CHUNKS

    def body(pi_hbm, x_hbm, out_ref, pi_ref, x_vmem, comm_ref,
             load_sems, pi_sem, send_sems, recv_sems):
        my_x = lax.axis_index("x")
        my_y = lax.axis_index("y")
        my_z = lax.axis_index("z")
        barrier_sem = pltpu.get_barrier_semaphore()

        pi_copy = pltpu.make_async_copy(pi_hbm, pi_ref, pi_sem)
        pi_copy.start()
        loads = []
        for i in range(N_CHUNKS):
            sl = pl.ds(i * rows, rows)
            cp = pltpu.make_async_copy(
                x_hbm.at[:, sl, :], x_vmem.at[:, sl, :], load_sems.at[i]
            )
            cp.start()
            loads.append(cp)
        pi_copy.wait()
        tgt_x = pi_ref[my_x]

        @pl.when(tgt_x == my_x)
        def _():
            for i in range(N_CHUNKS):
                sl = pl.ds(i * rows, rows)
                loads[i].wait()
                comm_ref[:, sl, :] = x_vmem[:, sl, :].astype(jnp.bfloat16)
            pltpu.make_async_copy(comm_ref, out_ref, send_sems.at[0]).start()
            pltpu.make_async_copy(comm_ref, out_ref, send_sems.at[0]).wait()

        @pl.when(tgt_x != my_x)
        def _():
            pl.semaphore_signal(
                barrier_sem,
                inc=1,
                device_id=(tgt_x, my_y, my_z),
                device_id_type=pl.DeviceIdType.MESH,
            )
            rdmas = []
            for i in range(N_CHUNKS):
                sl = pl.ds(i * rows, rows)
                loads[i].wait()
                comm_ref[:, sl, :] = x_vmem[:, sl, :].astype(jnp.bfloat16)
                if i == 0:
                    pl.semaphore_wait(barrier_sem, 1)
                rdma = pltpu.make_async_remote_copy(
                    src_ref=comm_ref.at[:, sl, :],
                    dst_ref=out_ref.at[:, sl, :],
                    send_sem=send_sems.at[i],
                    recv_sem=recv_sems.at[i],
                    device_id=(tgt_x, my_y, my_z),
                    device_id_type=pl.DeviceIdType.MESH,
                )
                rdma.start()
                rdmas.append(rdma)
            for rdma in rdmas:
                rdma.wait_send()
                rdma.wait_recv()

    return pl.pallas_call(
        body,
        out_shape=jax.ShapeDtypeStruct(x.shape, jnp.bfloat16),
        in_specs=[
            pl.BlockSpec(memory_space=pl.ANY),
            pl.BlockSpec(memory_space=pl.ANY),
        ],
        out_specs=pl.BlockSpec(memory_space=pl.ANY),
        scratch_shapes=[
            pltpu.SMEM((2,), jnp.int32),
            pltpu.VMEM(x.shape, x.dtype),
            pltpu.VMEM(x.shape, jnp.bfloat16),
            pltpu.SemaphoreType.DMA((N_CHUNKS,)),
            pltpu.SemaphoreType.DMA,
            pltpu.SemaphoreType.DMA((N_CHUNKS,)),
            pltpu.SemaphoreType.DMA((N_CHUNKS,)),
        ],
        compiler_params=pltpu.CompilerParams(collective_id=0),
    )(pi, x)


# device time: 12547 ns/iter; 1.0008x vs baseline; 1.0008x over previous
import jax
import jax.numpy as jnp
from jax import lax
from jax.experimental import pallas as pl
from jax.experimental.pallas import tpu as pltpu

N_CHUNKS = 4


def kernel(x, pi):
    _, m, n = x.shape
    rows = m // N_CHUNKS

    def body(pi_hbm, x_hbm, out_ref, pi_ref, x_vmem, comm_ref,
             load_sems, pi_sem, send_sems, recv_sems):
        my_x = lax.axis_index("x")
        my_y = lax.axis_index("y")
        my_z = lax.axis_index("z")
        barrier_sem = pltpu.get_barrier_semaphore()

        pi_copy = pltpu.make_async_copy(pi_hbm, pi_ref, pi_sem)
        pi_copy.start()
        loads = []
        for i in range(N_CHUNKS):
            sl = pl.ds(i * rows, rows)
            cp = pltpu.make_async_copy(
                x_hbm.at[:, sl, :], x_vmem.at[:, sl, :], load_sems.at[i]
            )
            cp.start()
            loads.append(cp)
        pi_copy.wait()
        tgt_x = pi_ref[my_x]

        @pl.when(tgt_x == my_x)
        def _():
            for i in range(N_CHUNKS):
                sl = pl.ds(i * rows, rows)
                loads[i].wait()
                comm_ref[:, sl, :] = x_vmem[:, sl, :].astype(jnp.bfloat16)
            pltpu.make_async_copy(comm_ref, out_ref, send_sems.at[0]).start()
            pltpu.make_async_copy(comm_ref, out_ref, send_sems.at[0]).wait()

        @pl.when(tgt_x != my_x)
        def _():
            pl.semaphore_signal(
                barrier_sem,
                inc=1,
                device_id=(tgt_x, my_y, my_z),
                device_id_type=pl.DeviceIdType.MESH,
            )
            rdmas = []
            for i in range(N_CHUNKS):
                sl = pl.ds(i * rows, rows)
                loads[i].wait()
                comm_ref[:, sl, :] = x_vmem[:, sl, :].astype(jnp.bfloat16)
                if i == 0:
                    pl.semaphore_wait(barrier_sem, 1)
                rdma = pltpu.make_async_remote_copy(
                    src_ref=comm_ref.at[:, sl, :],
                    dst_ref=out_ref.at[:, sl, :],
                    send_sem=send_sems.at[i],
                    recv_sem=recv_sems.at[i],
                    device_id=(tgt_x, my_y, my_z),
                    device_id_type=pl.DeviceIdType.MESH,
                )
                rdma.start()
                rdmas.append(rdma)
            for rdma in rdmas:
                rdma.wait_send()
                rdma.wait_recv()

    return pl.pallas_call(
        body,
        out_shape=jax.ShapeDtypeStruct(x.shape, jnp.bfloat16),
        in_specs=[
            pl.BlockSpec(memory_space=pltpu.MemorySpace.HBM),
            pl.BlockSpec(memory_space=pltpu.MemorySpace.HBM),
        ],
        out_specs=pl.BlockSpec(memory_space=pltpu.MemorySpace.HBM),
        scratch_shapes=[
            pltpu.SMEM((2,), jnp.int32),
            pltpu.VMEM(x.shape, x.dtype),
            pltpu.VMEM(x.shape, jnp.bfloat16),
            pltpu.SemaphoreType.DMA((N_CHUNKS,)),
            pltpu.SemaphoreType.DMA,
            pltpu.SemaphoreType.DMA((N_CHUNKS,)),
            pltpu.SemaphoreType.DMA((N_CHUNKS,)),
        ],
        compiler_params=pltpu.CompilerParams(collective_id=0),
    )(pi, x)


# device time: 12244 ns/iter; 1.0256x vs baseline; 1.0247x over previous
import jax
import jax.numpy as jnp
from jax import lax
from jax.experimental import pallas as pl
from jax.experimental.pallas import tpu as pltpu


def kernel(x, pi):
    def body(pi_ref, x_ref, out_ref, comm_ref, send_sem, recv_sem):
        my_x = lax.axis_index("x")
        my_y = lax.axis_index("y")
        my_z = lax.axis_index("z")
        tgt_x = pi_ref[my_x]
        barrier_sem = pltpu.get_barrier_semaphore()

        @pl.when(tgt_x == my_x)
        def _():
            out_ref[...] = x_ref[...].astype(jnp.bfloat16)

        @pl.when(tgt_x != my_x)
        def _():
            pl.semaphore_signal(
                barrier_sem,
                inc=1,
                device_id=(tgt_x, my_y, my_z),
                device_id_type=pl.DeviceIdType.MESH,
            )
            comm_ref[...] = x_ref[...].astype(jnp.bfloat16)
            pl.semaphore_wait(barrier_sem, 1)
            rdma = pltpu.make_async_remote_copy(
                src_ref=comm_ref,
                dst_ref=out_ref,
                send_sem=send_sem,
                recv_sem=recv_sem,
                device_id=(tgt_x, my_y, my_z),
                device_id_type=pl.DeviceIdType.MESH,
            )
            rdma.start()
            rdma.wait()

    return pl.pallas_call(
        body,
        out_shape=jax.ShapeDtypeStruct(x.shape, jnp.bfloat16),
        in_specs=[
            pl.BlockSpec(memory_space=pltpu.SMEM),
            pl.BlockSpec(memory_space=pltpu.VMEM),
        ],
        out_specs=pl.BlockSpec(memory_space=pltpu.VMEM),
        scratch_shapes=[
            pltpu.VMEM(x.shape, jnp.bfloat16),
            pltpu.SemaphoreType.DMA,
            pltpu.SemaphoreType.DMA,
        ],
        compiler_params=pltpu.CompilerParams(collective_id=0),
    )(pi, x)
